# baseline (speedup 1.0000x reference)
import math

import jax
import jax.numpy as jnp
from jax import lax
from jax.experimental import pallas as pl
from jax.experimental.pallas import tpu as pltpu
from jax.experimental.pallas import tpu_sc as plsc

_N = 512
_CUTMIN = 2
_CUTMAX = 32
_ALPHA = 1.0

_NCH = 96
_BANDS = 16
_HALVES = 2
_BAND_ROWS = _N // _BANDS
_CH_PER_W = _NCH // _HALVES
_SRC_ROWS = 64
_BAND_PX = _BAND_ROWS * _N
_QROWS = _BAND_ROWS // 4
_QVECS = _QROWS * _N // 16
_XQ = 32767.0


def _displacement_field():
    n = _N
    beta_sample = 0.5
    cut = int(beta_sample * (_CUTMAX + 1 - _CUTMIN) + _CUTMIN)
    c_ = cut + 1e-06
    log = math.log(c_)
    t1 = 1.0 / (math.pi * n ** 2 * log)
    t2 = 4.0 / (math.pi ** 3 * c_ ** 2 * log)
    t2 = max(t1, _ALPHA * t2)
    t = beta_sample * (t2 - t1) + t1

    x = jnp.linspace(0.0, 1.0, n, dtype=jnp.float32)
    k = jnp.arange(1, cut + 1, dtype=jnp.float32)
    i, j = jnp.meshgrid(k, k, indexing='ij')
    r = jnp.sqrt(i ** 2 + j ** 2)
    e = (r < cut + 0.5).astype(jnp.float32) / r
    s = jnp.sin(jnp.pi * x[:, None] * k[None, :])

    ku, kv = jax.random.split(jax.random.key(1))
    cu = jax.random.normal(ku, (cut, cut), dtype=jnp.float32) * e
    cv = jax.random.normal(kv, (cut, cut), dtype=jnp.float32) * e
    u = jnp.einsum('ij,xi,yj->yx', cu, s, s)
    v = jnp.einsum('ij,xi,yj->yx', cv, s, s)
    dx = (t ** 0.5) * u * n
    dy = (t ** 0.5) * v * n
    return dx, dy


def _gather_constants():
    n = _N
    dx, dy = _displacement_field()
    y, x = jnp.meshgrid(jnp.arange(n, dtype=jnp.float32),
                        jnp.arange(n, dtype=jnp.float32), indexing='ij')
    xn = jnp.clip(x - dx, 0.0, n - 1)
    yn = jnp.clip(y - dy, 0.0, n - 1)
    xf = jnp.minimum(jnp.floor(xn).astype(jnp.int32), n - 2)
    yf = jnp.minimum(jnp.floor(yn).astype(jnp.int32), n - 2)
    xv = xn - xf.astype(jnp.float32)
    yv = yn - yf.astype(jnp.float32)
    row = jnp.arange(n, dtype=jnp.int32)
    src_off = jnp.clip(_BAND_ROWS * (row // _BAND_ROWS) - 16, 0, n - _SRC_ROWS)
    i_tl = (yf - src_off[:, None]) * n + xf
    xq = jnp.round(xv * _XQ).astype(jnp.int32)
    packed = i_tl | (xq << 15)
    return packed.reshape(-1), yv.reshape(-1)


def _remap_body(img_hbm, pk_hbm, yv_hbm, out_hbm,
                pk_v, yv_v, src_a, src_b, out_q,
                sem_a, sem_b, sem_q0, sem_q1, sem_q2, sem_q3):
    j = lax.axis_index("s")
    h = lax.axis_index("c")
    r0 = j * _BAND_ROWS
    base_px = r0 * _N
    src_off = pl.multiple_of(jnp.clip(r0 - 16, 0, _N - _SRC_ROWS), 16)
    c0 = h * _CH_PER_W
    osems = (sem_q0, sem_q1, sem_q2, sem_q3)

    pltpu.sync_copy(pk_hbm.at[pl.ds(base_px, _BAND_PX)], pk_v)
    pltpu.sync_copy(yv_hbm.at[pl.ds(base_px, _BAND_PX)], yv_v)

    def in_desc(ci, buf, sem):
        c = c0 + ci
        return pltpu.make_async_copy(
            img_hbm.at[c // 3, c % 3, pl.ds(src_off, _SRC_ROWS), :], buf, sem)

    def out_desc(ci, q):
        c = c0 + ci
        return pltpu.make_async_copy(
            out_q.at[q],
            out_hbm.at[c // 3, c % 3, pl.ds(r0 + q * _QROWS, _QROWS), :],
            osems[q])

    def compute(ci, buf, first):
        for q in range(4):
            if not first:
                out_desc(ci, q).wait()

            @pl.loop(q * _QVECS, (q + 1) * _QVECS, unroll=4)
            def _inner(t):
                o = t * 16
                pk = pk_v[pl.ds(o, 16)]
                wy = yv_v[pl.ds(o, 16)]
                it = pk & 0x7FFF
                wx = (pk >> 15).astype(jnp.float32) * (1.0 / _XQ)
                r = it >> 9
                xx = it & (_N - 1)
                a00 = plsc.load_gather(buf, [r, xx])
                a01 = plsc.load_gather(buf, [r, xx + 1])
                a10 = plsc.load_gather(buf, [r + 1, xx])
                a11 = plsc.load_gather(buf, [r + 1, xx + 1])
                top = a00 + wx * (a01 - a00)
                bot = a10 + wx * (a11 - a10)
                out_q[q, (t >> 5) & (_QROWS - 1),
                      pl.ds((t & 31) * 16, 16)] = top + wy * (bot - top)

            out_desc(ci, q).start()

    in_desc(0, src_a, sem_a).start()
    in_desc(1, src_b, sem_b).start()

    in_desc(0, src_a, sem_a).wait()
    compute(0, src_a, True)
    in_desc(2, src_a, sem_a).start()
    in_desc(1, src_b, sem_b).wait()
    compute(1, src_b, False)
    in_desc(3, src_b, sem_b).start()

    @pl.loop(2, _CH_PER_W - 2, step=2)
    def _chan(ci):
        in_desc(ci, src_a, sem_a).wait()
        compute(ci, src_a, False)
        in_desc(ci + 2, src_a, sem_a).start()
        in_desc(ci + 1, src_b, sem_b).wait()
        compute(ci + 1, src_b, False)
        in_desc(ci + 3, src_b, sem_b).start()

    ci = _CH_PER_W - 2
    in_desc(ci, src_a, sem_a).wait()
    compute(ci, src_a, False)
    in_desc(ci + 1, src_b, sem_b).wait()
    compute(ci + 1, src_b, False)
    for q in range(4):
        out_desc(ci + 1, q).wait()


@jax.jit
def _diffeo_remap(img):
    pk, yv = _gather_constants()
    mesh = plsc.VectorSubcoreMesh(core_axis_name="c", subcore_axis_name="s")
    fn = pl.kernel(
        _remap_body,
        out_type=jax.ShapeDtypeStruct((32, 3, _N, _N), jnp.float32),
        mesh=mesh,
        compiler_params=pltpu.CompilerParams(needs_layout_passes=False),
        scratch_types=[
            pltpu.VMEM((_BAND_PX,), jnp.int32),
            pltpu.VMEM((_BAND_PX,), jnp.float32),
            pltpu.VMEM((_SRC_ROWS, _N), jnp.float32),
            pltpu.VMEM((_SRC_ROWS, _N), jnp.float32),
            pltpu.VMEM((4, _QROWS, _N), jnp.float32),
            pltpu.SemaphoreType.DMA,
            pltpu.SemaphoreType.DMA,
            pltpu.SemaphoreType.DMA,
            pltpu.SemaphoreType.DMA,
            pltpu.SemaphoreType.DMA,
            pltpu.SemaphoreType.DMA,
        ],
    )
    return fn(img, pk, yv)


def kernel(img):
    return _diffeo_remap(img)


# speedup vs baseline: 1.1316x; 1.1316x over previous
import math

import jax
import jax.numpy as jnp
from jax import lax
from jax.experimental import pallas as pl
from jax.experimental.pallas import tpu as pltpu
from jax.experimental.pallas import tpu_sc as plsc

_N = 512
_CUTMIN = 2
_CUTMAX = 32
_ALPHA = 1.0

_NCH = 96
_BANDS = 16
_HALVES = 2
_BAND_ROWS = _N // _BANDS
_CH_PER_W = _NCH // _HALVES
_SRC_ROWS = 64
_BAND_PX = _BAND_ROWS * _N
_QROWS = _BAND_ROWS // 4
_QVECS = _QROWS * _N // 16
_XQ = 32767.0


def _displacement_field():
    n = _N
    beta_sample = 0.5
    cut = int(beta_sample * (_CUTMAX + 1 - _CUTMIN) + _CUTMIN)
    c_ = cut + 1e-06
    log = math.log(c_)
    t1 = 1.0 / (math.pi * n ** 2 * log)
    t2 = 4.0 / (math.pi ** 3 * c_ ** 2 * log)
    t2 = max(t1, _ALPHA * t2)
    t = beta_sample * (t2 - t1) + t1

    x = jnp.linspace(0.0, 1.0, n, dtype=jnp.float32)
    k = jnp.arange(1, cut + 1, dtype=jnp.float32)
    i, j = jnp.meshgrid(k, k, indexing='ij')
    r = jnp.sqrt(i ** 2 + j ** 2)
    e = (r < cut + 0.5).astype(jnp.float32) / r
    s = jnp.sin(jnp.pi * x[:, None] * k[None, :])

    ku, kv = jax.random.split(jax.random.key(1))
    cu = jax.random.normal(ku, (cut, cut), dtype=jnp.float32) * e
    cv = jax.random.normal(kv, (cut, cut), dtype=jnp.float32) * e
    u = jnp.einsum('ij,xi,yj->yx', cu, s, s)
    v = jnp.einsum('ij,xi,yj->yx', cv, s, s)
    dx = (t ** 0.5) * u * n
    dy = (t ** 0.5) * v * n
    return dx, dy


def _gather_constants():
    n = _N
    dx, dy = _displacement_field()
    y, x = jnp.meshgrid(jnp.arange(n, dtype=jnp.float32),
                        jnp.arange(n, dtype=jnp.float32), indexing='ij')
    xn = jnp.clip(x - dx, 0.0, n - 1)
    yn = jnp.clip(y - dy, 0.0, n - 1)
    xf = jnp.minimum(jnp.floor(xn).astype(jnp.int32), n - 2)
    yf = jnp.minimum(jnp.floor(yn).astype(jnp.int32), n - 2)
    xv = xn - xf.astype(jnp.float32)
    yv = yn - yf.astype(jnp.float32)
    row = jnp.arange(n, dtype=jnp.int32)
    src_off = jnp.clip(_BAND_ROWS * (row // _BAND_ROWS) - 16, 0, n - _SRC_ROWS)
    i_tl = (yf - src_off[:, None]) * n + xf
    xq = jnp.round(xv * _XQ).astype(jnp.int32)
    packed = i_tl | (xq << 15)
    return packed.reshape(-1), yv.reshape(-1)


def _remap_body(img_hbm, pk_hbm, yv_hbm, out_hbm,
                pk_v, yv_v, src_a, src_b, out_q,
                sem_a, sem_b, sem_q0, sem_q1, sem_q2, sem_q3):
    j = lax.axis_index("s")
    h = lax.axis_index("c")
    r0 = j * _BAND_ROWS
    base_px = r0 * _N
    src_off = pl.multiple_of(jnp.clip(r0 - 16, 0, _N - _SRC_ROWS), 16)
    c0 = h * _CH_PER_W
    osems = (sem_q0, sem_q1, sem_q2, sem_q3)

    pltpu.sync_copy(pk_hbm.at[pl.ds(base_px, _BAND_PX)], pk_v)
    pltpu.sync_copy(yv_hbm.at[pl.ds(base_px, _BAND_PX)], yv_v)

    def in_desc(ci, buf, sem):
        return pltpu.make_async_copy(
            img_hbm.at[c0 + ci, pl.ds(src_off * _N, _SRC_ROWS * _N)], buf, sem)

    def out_desc(ci, q):
        c = c0 + ci
        return pltpu.make_async_copy(
            out_q.at[q],
            out_hbm.at[c // 3, c % 3, pl.ds(r0 + q * _QROWS, _QROWS), :],
            osems[q])

    def compute(ci, buf, first):
        flat = buf
        for q in range(4):
            if not first:
                out_desc(ci, q).wait()

            @pl.loop(q * _QVECS, (q + 1) * _QVECS, unroll=4)
            def _inner(t):
                o = t * 16
                pk = pk_v[pl.ds(o, 16)]
                wy = yv_v[pl.ds(o, 16)]
                it = pk & 0x7FFF
                wx = (pk >> 15).astype(jnp.float32) * (1.0 / _XQ)
                a00 = plsc.load_gather(flat, [it])
                a01 = plsc.load_gather(flat, [it + 1])
                a10 = plsc.load_gather(flat, [it + _N])
                a11 = plsc.load_gather(flat, [it + (_N + 1)])
                top = a00 + wx * (a01 - a00)
                bot = a10 + wx * (a11 - a10)
                out_q[q, (t >> 5) & (_QROWS - 1),
                      pl.ds((t & 31) * 16, 16)] = top + wy * (bot - top)

            out_desc(ci, q).start()

    in_desc(0, src_a, sem_a).start()
    in_desc(1, src_b, sem_b).start()

    in_desc(0, src_a, sem_a).wait()
    compute(0, src_a, True)
    in_desc(2, src_a, sem_a).start()
    in_desc(1, src_b, sem_b).wait()
    compute(1, src_b, False)
    in_desc(3, src_b, sem_b).start()

    @pl.loop(2, _CH_PER_W - 2, step=2)
    def _chan(ci):
        in_desc(ci, src_a, sem_a).wait()
        compute(ci, src_a, False)
        in_desc(ci + 2, src_a, sem_a).start()
        in_desc(ci + 1, src_b, sem_b).wait()
        compute(ci + 1, src_b, False)
        in_desc(ci + 3, src_b, sem_b).start()

    ci = _CH_PER_W - 2
    in_desc(ci, src_a, sem_a).wait()
    compute(ci, src_a, False)
    in_desc(ci + 1, src_b, sem_b).wait()
    compute(ci + 1, src_b, False)
    for q in range(4):
        out_desc(ci + 1, q).wait()


@jax.jit
def _diffeo_remap(img):
    pk, yv = _gather_constants()
    mesh = plsc.VectorSubcoreMesh(core_axis_name="c", subcore_axis_name="s")
    fn = pl.kernel(
        _remap_body,
        out_type=jax.ShapeDtypeStruct((32, 3, _N, _N), jnp.float32),
        mesh=mesh,
        compiler_params=pltpu.CompilerParams(needs_layout_passes=False),
        scratch_types=[
            pltpu.VMEM((_BAND_PX,), jnp.int32),
            pltpu.VMEM((_BAND_PX,), jnp.float32),
            pltpu.VMEM((_SRC_ROWS * _N,), jnp.float32),
            pltpu.VMEM((_SRC_ROWS * _N,), jnp.float32),
            pltpu.VMEM((4, _QROWS, _N), jnp.float32),
            pltpu.SemaphoreType.DMA,
            pltpu.SemaphoreType.DMA,
            pltpu.SemaphoreType.DMA,
            pltpu.SemaphoreType.DMA,
            pltpu.SemaphoreType.DMA,
            pltpu.SemaphoreType.DMA,
        ],
    )
    return fn(img.reshape(_NCH, _N * _N), pk, yv)


def kernel(img):
    return _diffeo_remap(img)
